# Initial kernel scaffold; baseline (speedup 1.0000x reference)
import jax
import jax.numpy as jnp
from jax.experimental import pallas as pl


def kernel(x_prev, x_same, x_next, edge_index, gamma, beta, W, b):
    raise NotImplementedError("write your pallas kernel here")


# speedup vs baseline: 27.1466x; 27.1466x over previous
import functools

import jax
import jax.numpy as jnp
from jax import lax
from jax.experimental import pallas as pl
from jax.experimental.pallas import tpu as pltpu
from jax.experimental.pallas import tpu_sc as plsc

_NC = 2
_NS = 16
_NW = _NC * _NS
_CH = 80


def _sc_hist_body(NP, SL, KCH, dst_hbm, ones_hbm, zeros_hbm, hist_out,
                  dst_v, ones_v, hist_sh):
    c = lax.axis_index("c")
    s = lax.axis_index("s")
    w = s * _NC + c
    pltpu.sync_copy(ones_hbm, ones_v)
    pltpu.sync_copy(zeros_hbm.at[pl.ds(s * SL, SL)],
                    hist_sh.at[pl.ds(s * SL, SL)])
    pltpu.sync_copy(dst_hbm.at[w], dst_v)
    plsc.subcore_barrier()

    def body(i, carry):
        pltpu.sync_copy(ones_v, hist_sh.at[dst_v.at[i]], add=True)
        return carry

    lax.fori_loop(0, KCH, body, 0)
    plsc.subcore_barrier()
    pltpu.sync_copy(hist_sh.at[pl.ds(s * SL, SL)],
                    hist_out.at[c, pl.ds(s * SL, SL)])


def _sc_agg_body(NP, SL, KCH, OUT, y_hbm, src_hbm, dst_hbm, zeros_hbm,
                 agg_out, src_v, dst_v, rows_v, agg_sh, sem):
    c = lax.axis_index("c")
    s = lax.axis_index("s")
    w = s * _NC + c
    pltpu.sync_copy(zeros_hbm.at[pl.ds(s * SL, SL)],
                    agg_sh.at[pl.ds(s * SL, SL)])
    pltpu.sync_copy(src_hbm.at[w], src_v)
    pltpu.sync_copy(dst_hbm.at[w], dst_v)
    plsc.subcore_barrier()

    def body(i, carry):
        pltpu.async_copy(y_hbm.at[src_v.at[i]], rows_v, sem).wait()
        pltpu.sync_copy(rows_v, agg_sh.at[dst_v.at[i]], add=True)
        return carry

    lax.fori_loop(0, KCH, body, 0)
    plsc.subcore_barrier()
    pltpu.sync_copy(agg_sh.at[pl.ds(s * SL, SL)],
                    agg_out.at[c, pl.ds(s * SL, SL)])


def _tc_pre_body(D, xp_ref, xn_ref, g_ref, be_ref, W_ref, hist_ref, y_ref):
    g = g_ref[0:1, :]
    be = be_ref[0:1, :]

    def ln(x):
        mu = jnp.mean(x, axis=-1, keepdims=True)
        xc = x - mu
        var = jnp.mean(xc * xc, axis=-1, keepdims=True)
        return xc * lax.rsqrt(var + 1e-5) * g + be

    a = ln(xp_ref[...])
    b2 = ln(xn_ref[...])
    xw = (jnp.dot(a, W_ref[0:D, :], preferred_element_type=jnp.float32)
          + jnp.dot(b2, W_ref[D:2 * D, :], preferred_element_type=jnp.float32))
    h = hist_ref[...]
    deg = 1.0 + h[0, :, 0:1] + h[1, :, 0:1]
    y_ref[...] = xw * lax.rsqrt(deg)


def _tc_post_body(agg_ref, y_ref, hist_ref, b_ref, out_ref):
    h = hist_ref[...]
    deg = 1.0 + h[0, :, 0:1] + h[1, :, 0:1]
    dinv = lax.rsqrt(deg)
    acc = agg_ref[0] + agg_ref[1] + y_ref[...]
    out_ref[...] = acc * dinv + b_ref[0:1, :]


def kernel(x_prev, x_same, x_next, edge_index, gamma, beta, W, b):
    N, D = x_prev.shape
    OUT = W.shape[1]
    E = edge_index.shape[1]
    KCH = E // (_NW * _CH)
    assert _NW * _CH * KCH == E
    NP = ((N + 127) // 128) * 128
    SL = NP // _NS
    RB = 1000
    GRID = N // RB

    f32 = jnp.float32
    src3 = edge_index[0].reshape(_NW, KCH, _CH)
    dst3 = edge_index[1].reshape(_NW, KCH, _CH)
    ones8 = jnp.ones((_CH, 8), f32)
    zeros8 = jnp.zeros((NP, 8), f32)
    zerosR = jnp.zeros((NP, OUT), f32)
    g2 = jnp.broadcast_to(gamma.reshape(1, D), (8, D))
    be2 = jnp.broadcast_to(beta.reshape(1, D), (8, D))
    b2 = jnp.broadcast_to(b.reshape(1, OUT), (8, OUT))

    mesh = plsc.VectorSubcoreMesh(core_axis_name="c", subcore_axis_name="s",
                                  num_cores=_NC, num_subcores=_NS)

    hist = pl.kernel(
        functools.partial(_sc_hist_body, NP, SL, KCH),
        out_type=jax.ShapeDtypeStruct((_NC, NP, 8), f32),
        mesh=mesh,
        compiler_params=pltpu.CompilerParams(use_tc_tiling_on_sc=False),
        scratch_types=[
            pltpu.VMEM((KCH, _CH), jnp.int32),
            pltpu.VMEM((_CH, 8), f32),
            pltpu.VMEM_SHARED((NP, 8), f32),
        ],
    )(dst3, ones8, zeros8)

    y = pl.pallas_call(
        functools.partial(_tc_pre_body, D),
        grid=(GRID,),
        in_specs=[
            pl.BlockSpec((RB, D), lambda i: (i, 0)),
            pl.BlockSpec((RB, D), lambda i: (i, 0)),
            pl.BlockSpec((8, D), lambda i: (0, 0)),
            pl.BlockSpec((8, D), lambda i: (0, 0)),
            pl.BlockSpec((2 * D, OUT), lambda i: (0, 0)),
            pl.BlockSpec((_NC, RB, 8), lambda i: (0, i, 0)),
        ],
        out_specs=pl.BlockSpec((RB, OUT), lambda i: (i, 0)),
        out_shape=jax.ShapeDtypeStruct((N, OUT), f32),
    )(x_prev, x_next, g2, be2, W, hist)

    agg = pl.kernel(
        functools.partial(_sc_agg_body, NP, SL, KCH, OUT),
        out_type=jax.ShapeDtypeStruct((_NC, NP, OUT), f32),
        mesh=mesh,
        compiler_params=pltpu.CompilerParams(use_tc_tiling_on_sc=False),
        scratch_types=[
            pltpu.VMEM((KCH, _CH), jnp.int32),
            pltpu.VMEM((KCH, _CH), jnp.int32),
            pltpu.VMEM((_CH, OUT), f32),
            pltpu.VMEM_SHARED((NP, OUT), f32),
            pltpu.SemaphoreType.DMA,
        ],
    )(y, src3, dst3, zerosR)

    out = pl.pallas_call(
        _tc_post_body,
        grid=(GRID,),
        in_specs=[
            pl.BlockSpec((_NC, RB, OUT), lambda i: (0, i, 0)),
            pl.BlockSpec((RB, OUT), lambda i: (i, 0)),
            pl.BlockSpec((_NC, RB, 8), lambda i: (0, i, 0)),
            pl.BlockSpec((8, OUT), lambda i: (0, 0)),
        ],
        out_specs=pl.BlockSpec((RB, OUT), lambda i: (i, 0)),
        out_shape=jax.ShapeDtypeStruct((N, OUT), f32),
    )(agg, y, hist, b2)

    return out


# speedup vs baseline: 30.7407x; 1.1324x over previous
import functools

import jax
import jax.numpy as jnp
from jax import lax
from jax.experimental import pallas as pl
from jax.experimental.pallas import tpu as pltpu
from jax.experimental.pallas import tpu_sc as plsc

_NC = 2
_NS = 16
_NW = _NC * _NS
_CH = 80


def _sc_hist_body(NP, SL, KCH, dst_hbm, ones_hbm, zeros_hbm, hist_out,
                  dst_v, ones_v, hist_sh):
    c = lax.axis_index("c")
    s = lax.axis_index("s")
    w = s * _NC + c
    pltpu.sync_copy(ones_hbm, ones_v)
    pltpu.sync_copy(zeros_hbm.at[pl.ds(s * SL, SL)],
                    hist_sh.at[pl.ds(s * SL, SL)])
    pltpu.sync_copy(dst_hbm.at[w], dst_v)
    plsc.subcore_barrier()

    def body(i, carry):
        pltpu.sync_copy(ones_v, hist_sh.at[dst_v.at[i]], add=True)
        return carry

    lax.fori_loop(0, KCH, body, 0)
    plsc.subcore_barrier()
    pltpu.sync_copy(hist_sh.at[pl.ds(s * SL, SL)],
                    hist_out.at[c, pl.ds(s * SL, SL)])


def _sc_agg_body(NP, SL, KCH, OUT, y_hbm, src_hbm, dst_hbm, zeros_hbm,
                 agg_out, src_v, dst_v, rows_a, rows_b, agg_sh, sem_a, sem_b):
    c = lax.axis_index("c")
    s = lax.axis_index("s")
    w = s * _NC + c
    pltpu.sync_copy(zeros_hbm.at[pl.ds(s * SL, SL)],
                    agg_sh.at[pl.ds(s * SL, SL)])
    pltpu.sync_copy(src_hbm.at[w], src_v)
    pltpu.sync_copy(dst_hbm.at[w], dst_v)
    plsc.subcore_barrier()

    bufs = (rows_a, rows_b)
    sems = (sem_a, sem_b)
    pltpu.async_copy(y_hbm.at[src_v.at[0]], rows_a, sem_a)

    def step(i, buf, sem, obuf, osem):
        pltpu.make_async_copy(y_hbm.at[src_v.at[i]], buf, sem).wait()

        @pl.when(i + 1 < KCH)
        def _():
            pltpu.async_copy(y_hbm.at[src_v.at[i + 1]], obuf, osem)

        pltpu.sync_copy(buf, agg_sh.at[dst_v.at[i]], add=True)

    def body(i, carry):
        @pl.when(lax.rem(i, 2) == 0)
        def _():
            step(i, rows_a, sem_a, rows_b, sem_b)

        @pl.when(lax.rem(i, 2) == 1)
        def _():
            step(i, rows_b, sem_b, rows_a, sem_a)

        return carry

    lax.fori_loop(0, KCH, body, 0)
    plsc.subcore_barrier()
    pltpu.sync_copy(agg_sh.at[pl.ds(s * SL, SL)],
                    agg_out.at[c, pl.ds(s * SL, SL)])


def _tc_pre_body(D, xp_ref, xn_ref, g_ref, be_ref, W_ref, hist_ref, y_ref):
    g = g_ref[0:1, :]
    be = be_ref[0:1, :]

    def ln(x):
        mu = jnp.mean(x, axis=-1, keepdims=True)
        xc = x - mu
        var = jnp.mean(xc * xc, axis=-1, keepdims=True)
        return xc * lax.rsqrt(var + 1e-5) * g + be

    a = ln(xp_ref[...])
    b2 = ln(xn_ref[...])
    xw = (jnp.dot(a, W_ref[0:D, :], preferred_element_type=jnp.float32)
          + jnp.dot(b2, W_ref[D:2 * D, :], preferred_element_type=jnp.float32))
    h = hist_ref[...]
    deg = 1.0 + h[0, :, 0:1] + h[1, :, 0:1]
    y_ref[...] = xw * lax.rsqrt(deg)


def _tc_post_body(agg_ref, y_ref, hist_ref, b_ref, out_ref):
    h = hist_ref[...]
    deg = 1.0 + h[0, :, 0:1] + h[1, :, 0:1]
    dinv = lax.rsqrt(deg)
    acc = agg_ref[0] + agg_ref[1] + y_ref[...]
    out_ref[...] = acc * dinv + b_ref[0:1, :]


def kernel(x_prev, x_same, x_next, edge_index, gamma, beta, W, b):
    N, D = x_prev.shape
    OUT = W.shape[1]
    E = edge_index.shape[1]
    KCH = E // (_NW * _CH)
    assert _NW * _CH * KCH == E
    NP = ((N + 127) // 128) * 128
    SL = NP // _NS
    RB = 1000
    GRID = N // RB

    f32 = jnp.float32
    src3 = edge_index[0].reshape(_NW, KCH, _CH)
    dst3 = edge_index[1].reshape(_NW, KCH, _CH)
    ones8 = jnp.ones((_CH, 8), f32)
    zeros8 = jnp.zeros((NP, 8), f32)
    zerosR = jnp.zeros((NP, OUT), f32)
    g2 = jnp.broadcast_to(gamma.reshape(1, D), (8, D))
    be2 = jnp.broadcast_to(beta.reshape(1, D), (8, D))
    b2 = jnp.broadcast_to(b.reshape(1, OUT), (8, OUT))

    mesh = plsc.VectorSubcoreMesh(core_axis_name="c", subcore_axis_name="s",
                                  num_cores=_NC, num_subcores=_NS)

    hist = pl.kernel(
        functools.partial(_sc_hist_body, NP, SL, KCH),
        out_type=jax.ShapeDtypeStruct((_NC, NP, 8), f32),
        mesh=mesh,
        compiler_params=pltpu.CompilerParams(use_tc_tiling_on_sc=False),
        scratch_types=[
            pltpu.VMEM((KCH, _CH), jnp.int32),
            pltpu.VMEM((_CH, 8), f32),
            pltpu.VMEM_SHARED((NP, 8), f32),
        ],
    )(dst3, ones8, zeros8)

    y = pl.pallas_call(
        functools.partial(_tc_pre_body, D),
        grid=(GRID,),
        in_specs=[
            pl.BlockSpec((RB, D), lambda i: (i, 0)),
            pl.BlockSpec((RB, D), lambda i: (i, 0)),
            pl.BlockSpec((8, D), lambda i: (0, 0)),
            pl.BlockSpec((8, D), lambda i: (0, 0)),
            pl.BlockSpec((2 * D, OUT), lambda i: (0, 0)),
            pl.BlockSpec((_NC, RB, 8), lambda i: (0, i, 0)),
        ],
        out_specs=pl.BlockSpec((RB, OUT), lambda i: (i, 0)),
        out_shape=jax.ShapeDtypeStruct((N, OUT), f32),
    )(x_prev, x_next, g2, be2, W, hist)

    agg = pl.kernel(
        functools.partial(_sc_agg_body, NP, SL, KCH, OUT),
        out_type=jax.ShapeDtypeStruct((_NC, NP, OUT), f32),
        mesh=mesh,
        compiler_params=pltpu.CompilerParams(use_tc_tiling_on_sc=False),
        scratch_types=[
            pltpu.VMEM((KCH, _CH), jnp.int32),
            pltpu.VMEM((KCH, _CH), jnp.int32),
            pltpu.VMEM((_CH, OUT), f32),
            pltpu.VMEM((_CH, OUT), f32),
            pltpu.VMEM_SHARED((NP, OUT), f32),
            pltpu.SemaphoreType.DMA,
            pltpu.SemaphoreType.DMA,
        ],
    )(y, src3, dst3, zerosR)

    out = pl.pallas_call(
        _tc_post_body,
        grid=(GRID,),
        in_specs=[
            pl.BlockSpec((_NC, RB, OUT), lambda i: (0, i, 0)),
            pl.BlockSpec((RB, OUT), lambda i: (i, 0)),
            pl.BlockSpec((_NC, RB, 8), lambda i: (0, i, 0)),
            pl.BlockSpec((8, OUT), lambda i: (0, 0)),
        ],
        out_specs=pl.BlockSpec((RB, OUT), lambda i: (i, 0)),
        out_shape=jax.ShapeDtypeStruct((N, OUT), f32),
    )(agg, y, hist, b2)

    return out


# speedup vs baseline: 37.6288x; 1.2241x over previous
import functools

import jax
import jax.numpy as jnp
from jax import lax
from jax.experimental import pallas as pl
from jax.experimental.pallas import tpu as pltpu
from jax.experimental.pallas import tpu_sc as plsc

_NC = 2
_NS = 16
_NW = _NC * _NS
_CH = 80


def _sc_hist_body(NP, SL, KCH, dst_hbm, ones_hbm, zeros_hbm, hist_out,
                  dst_v, ones_v, hist_sh):
    c = lax.axis_index("c")
    s = lax.axis_index("s")
    w = s * _NC + c
    pltpu.sync_copy(ones_hbm, ones_v)
    pltpu.sync_copy(zeros_hbm.at[pl.ds(s * SL, SL)],
                    hist_sh.at[pl.ds(s * SL, SL)])
    pltpu.sync_copy(dst_hbm.at[w], dst_v)
    plsc.subcore_barrier()

    def body(i, carry):
        pltpu.sync_copy(ones_v, hist_sh.at[dst_v.at[i]], add=True)
        return carry

    lax.fori_loop(0, KCH, body, 0)
    plsc.subcore_barrier()
    pltpu.sync_copy(hist_sh.at[pl.ds(s * SL, SL)],
                    hist_out.at[c, pl.ds(s * SL, SL)])


def _sc_agg_body(NP, SL, KCH, OUT, y_hbm, src_hbm, dst_hbm, zeros_hbm,
                 agg_out, src_v, dst_v, r0, r1, r2, r3, agg_sh,
                 g0, g1, g2, g3, s0, s1, s2, s3):
    c = lax.axis_index("c")
    s = lax.axis_index("s")
    w = s * _NC + c
    pltpu.sync_copy(zeros_hbm.at[pl.ds(s * SL, SL)],
                    agg_sh.at[pl.ds(s * SL, SL)])
    pltpu.sync_copy(src_hbm.at[w], src_v)
    pltpu.sync_copy(dst_hbm.at[w], dst_v)
    plsc.subcore_barrier()

    bufs = (r0, r1, r2, r3)
    gsem = (g0, g1, g2, g3)
    ssem = (s0, s1, s2, s3)

    pltpu.async_copy(y_hbm.at[src_v.at[0]], r0, g0)
    pltpu.async_copy(y_hbm.at[src_v.at[1]], r1, g1)

    def step(i, b, b2):
        pltpu.make_async_copy(y_hbm.at[src_v.at[i]], bufs[b], gsem[b]).wait()
        pltpu.async_copy(bufs[b], agg_sh.at[dst_v.at[i]], ssem[b], add=True)

        @pl.when((i >= 2) & (i + 2 < KCH))
        def _():
            pltpu.make_async_copy(bufs[b2], agg_sh.at[dst_v.at[i - 2]],
                                  ssem[b2]).wait()
            pltpu.async_copy(y_hbm.at[src_v.at[i + 2]], bufs[b2], gsem[b2])

        @pl.when((i < 2) & (i + 2 < KCH))
        def _():
            pltpu.async_copy(y_hbm.at[src_v.at[i + 2]], bufs[b2], gsem[b2])

    def body(i, carry):
        for r in range(4):
            @pl.when(lax.rem(i, 4) == r)
            def _(r=r):
                step(i, r, (r + 2) % 4)
        return carry

    lax.fori_loop(0, KCH, body, 0)
    for j in range(max(0, KCH - 4), KCH):
        pltpu.make_async_copy(bufs[j % 4], agg_sh.at[dst_v.at[j]],
                              ssem[j % 4]).wait()
    plsc.subcore_barrier()
    pltpu.sync_copy(agg_sh.at[pl.ds(s * SL, SL)],
                    agg_out.at[c, pl.ds(s * SL, SL)])


def _tc_pre_body(D, xp_ref, xn_ref, g_ref, be_ref, W_ref, hist_ref, y_ref):
    g = g_ref[0:1, :]
    be = be_ref[0:1, :]

    def ln(x):
        mu = jnp.mean(x, axis=-1, keepdims=True)
        xc = x - mu
        var = jnp.mean(xc * xc, axis=-1, keepdims=True)
        return xc * lax.rsqrt(var + 1e-5) * g + be

    a = ln(xp_ref[...])
    b2 = ln(xn_ref[...])
    xw = (jnp.dot(a, W_ref[0:D, :], preferred_element_type=jnp.float32)
          + jnp.dot(b2, W_ref[D:2 * D, :], preferred_element_type=jnp.float32))
    h = hist_ref[...]
    deg = 1.0 + h[0, :, 0:1] + h[1, :, 0:1]
    y_ref[...] = xw * lax.rsqrt(deg)


def _tc_post_body(agg_ref, y_ref, hist_ref, b_ref, out_ref):
    h = hist_ref[...]
    deg = 1.0 + h[0, :, 0:1] + h[1, :, 0:1]
    dinv = lax.rsqrt(deg)
    acc = agg_ref[0] + agg_ref[1] + y_ref[...]
    out_ref[...] = acc * dinv + b_ref[0:1, :]


def kernel(x_prev, x_same, x_next, edge_index, gamma, beta, W, b):
    N, D = x_prev.shape
    OUT = W.shape[1]
    E = edge_index.shape[1]
    KCH = E // (_NW * _CH)
    assert _NW * _CH * KCH == E
    NP = ((N + 127) // 128) * 128
    SL = NP // _NS
    RB = 1000
    GRID = N // RB

    f32 = jnp.float32
    src3 = edge_index[0].reshape(_NW, KCH, _CH)
    dst3 = edge_index[1].reshape(_NW, KCH, _CH)
    ones8 = jnp.ones((_CH, 8), f32)
    zeros8 = jnp.zeros((NP, 8), f32)
    zerosR = jnp.zeros((NP, OUT), f32)
    g2 = jnp.broadcast_to(gamma.reshape(1, D), (8, D))
    be2 = jnp.broadcast_to(beta.reshape(1, D), (8, D))
    b2 = jnp.broadcast_to(b.reshape(1, OUT), (8, OUT))

    mesh = plsc.VectorSubcoreMesh(core_axis_name="c", subcore_axis_name="s",
                                  num_cores=_NC, num_subcores=_NS)

    hist = pl.kernel(
        functools.partial(_sc_hist_body, NP, SL, KCH),
        out_type=jax.ShapeDtypeStruct((_NC, NP, 8), f32),
        mesh=mesh,
        compiler_params=pltpu.CompilerParams(use_tc_tiling_on_sc=False),
        scratch_types=[
            pltpu.VMEM((KCH, _CH), jnp.int32),
            pltpu.VMEM((_CH, 8), f32),
            pltpu.VMEM_SHARED((NP, 8), f32),
        ],
    )(dst3, ones8, zeros8)

    y = pl.pallas_call(
        functools.partial(_tc_pre_body, D),
        grid=(GRID,),
        in_specs=[
            pl.BlockSpec((RB, D), lambda i: (i, 0)),
            pl.BlockSpec((RB, D), lambda i: (i, 0)),
            pl.BlockSpec((8, D), lambda i: (0, 0)),
            pl.BlockSpec((8, D), lambda i: (0, 0)),
            pl.BlockSpec((2 * D, OUT), lambda i: (0, 0)),
            pl.BlockSpec((_NC, RB, 8), lambda i: (0, i, 0)),
        ],
        out_specs=pl.BlockSpec((RB, OUT), lambda i: (i, 0)),
        out_shape=jax.ShapeDtypeStruct((N, OUT), f32),
    )(x_prev, x_next, g2, be2, W, hist)

    agg = pl.kernel(
        functools.partial(_sc_agg_body, NP, SL, KCH, OUT),
        out_type=jax.ShapeDtypeStruct((_NC, NP, OUT), f32),
        mesh=mesh,
        compiler_params=pltpu.CompilerParams(use_tc_tiling_on_sc=False),
        scratch_types=[
            pltpu.VMEM((KCH, _CH), jnp.int32),
            pltpu.VMEM((KCH, _CH), jnp.int32),
            pltpu.VMEM((_CH, OUT), f32),
            pltpu.VMEM((_CH, OUT), f32),
            pltpu.VMEM((_CH, OUT), f32),
            pltpu.VMEM((_CH, OUT), f32),
            pltpu.VMEM_SHARED((NP, OUT), f32),
        ] + [pltpu.SemaphoreType.DMA] * 8,
    )(y, src3, dst3, zerosR)

    out = pl.pallas_call(
        _tc_post_body,
        grid=(GRID,),
        in_specs=[
            pl.BlockSpec((_NC, RB, OUT), lambda i: (0, i, 0)),
            pl.BlockSpec((RB, OUT), lambda i: (i, 0)),
            pl.BlockSpec((_NC, RB, 8), lambda i: (0, i, 0)),
            pl.BlockSpec((8, OUT), lambda i: (0, 0)),
        ],
        out_specs=pl.BlockSpec((RB, OUT), lambda i: (i, 0)),
        out_shape=jax.ShapeDtypeStruct((N, OUT), f32),
    )(agg, y, hist, b2)

    return out
